# baseline (speedup 1.0000x reference)
import jax
import jax.numpy as jnp
from jax import lax
from jax.experimental import pallas as pl
from jax.experimental.pallas import tpu as pltpu
from jax.experimental.pallas import tpu_sc as plsc

B = 4096
NUM = 13
CAT = 26
FIELDS = 39
VOCAB = 100000
D = 16
H = 400
VG = VOCAB // 16

NC, NS = 2, 16
NW = NC * NS
BPW = B // NW
BCAT = B * CAT
PER_W = BCAT // NW
RPF = BPW * D

BB = 1024


def _sc_gather_e2(qidx, lanes, t2, nb):
  mesh = plsc.VectorSubcoreMesh(core_axis_name="c", subcore_axis_name="s",
                                num_cores=NC, num_subcores=NS)
  bpw = nb // NW
  rpf = bpw * D

  def body(qidx_hbm, lane_hbm, t2_hbm, ecatT_hbm, qidx_v, lane_v, w16, colbuf,
           sem0, sem1):
    li = lax.iota(jnp.int32, 16)
    wid = lax.axis_index("s") * NC + lax.axis_index("c")
    b0 = wid * bpw
    sems = (sem0, sem1)

    def fire(f, p):
      pltpu.sync_copy(qidx_hbm.at[f, pl.ds(b0 * D, rpf)],
                      qidx_v.at[pl.ds(p * rpf, rpf)])
      pltpu.sync_copy(lane_hbm.at[f, pl.ds(b0, bpw)],
                      lane_v.at[pl.ds(p * bpw, bpw)])

      def chunk(c, _):
        pltpu.async_copy(t2_hbm.at[qidx_v.at[pl.ds(p * rpf + c * 128, 128)]],
                         w16.at[pl.ds(p * rpf + c * 128, 128)], sems[p])
        return 0

      lax.fori_loop(0, rpf // 128, chunk, 0)

    def drain_sel_out(f, p):
      pltpu.make_async_copy(t2_hbm.at[pl.ds(0, rpf)],
                            w16.at[pl.ds(p * rpf, rpf)], sems[p]).wait()

      def select(t, _):
        lane_chunk = lane_v[pl.ds(p * bpw + t * 16, 16)]
        rbase = p * rpf + 256 * t + li * 16
        for d in range(D):
          vals = plsc.load_gather(w16, [rbase + d, lane_chunk])
          colbuf[d, pl.ds(t * 16, 16)] = vals
        return 0

      lax.fori_loop(0, bpw // 16, select, 0)
      pltpu.sync_copy(colbuf, ecatT_hbm.at[pl.ds(16 * f, 16), pl.ds(b0, bpw)])

    def pair(k, _):
      f0 = 2 * k
      fire(f0, 0)

      @pl.when(k > 0)
      def _():
        drain_sel_out(f0 - 1, 1)

      fire(f0 + 1, 1)
      drain_sel_out(f0, 0)
      return 0

    lax.fori_loop(0, CAT // 2, pair, 0)
    drain_sel_out(CAT - 1, 1)

  return pl.kernel(
      body,
      out_type=jax.ShapeDtypeStruct((CAT * D, nb), jnp.float32),
      mesh=mesh,
      compiler_params=pltpu.CompilerParams(use_tc_tiling_on_sc=False,
                                           needs_layout_passes=False),
      scratch_types=[pltpu.VMEM((2 * rpf,), jnp.int32),
                     pltpu.VMEM((2 * bpw,), jnp.int32),
                     pltpu.VMEM((2 * rpf, 16), jnp.float32),
                     pltpu.VMEM((D, bpw), jnp.float32),
                     pltpu.SemaphoreType.DMA,
                     pltpu.SemaphoreType.DMA],
  )(qidx, lanes, t2)


def _sc_gather_e1(flat_idx, row_idx, emb1_rows):
  mesh = plsc.VectorSubcoreMesh(core_axis_name="c", subcore_axis_name="s",
                                num_cores=NC, num_subcores=NS)
  CH = 128
  NCH = PER_W // CH

  def body(idx_hbm, ridx_hbm, t1_hbm, w_hbm, idx_v, ridx_v, w16_v, w_v, sem):
    wid = lax.axis_index("s") * NC + lax.axis_index("c")
    base = wid * PER_W
    pltpu.sync_copy(idx_hbm.at[pl.ds(base, PER_W)], idx_v)
    pltpu.sync_copy(ridx_hbm.at[pl.ds(base, PER_W)], ridx_v)
    copies = []
    for c in range(NCH):
      sl = pl.ds(c * CH, CH)
      copies.append(pltpu.async_copy(t1_hbm.at[ridx_v.at[sl]], w16_v.at[sl], sem))
    for c in copies:
      c.wait()

    def pick(t, _):
      sl = pl.ds(t * 16, 16)
      lane = idx_v[sl] & 15
      row = t * 16 + lax.iota(jnp.int32, 16)
      w_v[sl] = plsc.load_gather(w16_v, [row, lane])
      return 0

    lax.fori_loop(0, PER_W // 16, pick, 0)
    pltpu.sync_copy(w_v, w_hbm.at[pl.ds(base, PER_W)])

  return pl.kernel(
      body,
      out_type=jax.ShapeDtypeStruct((BCAT,), jnp.float32),
      mesh=mesh,
      compiler_params=pltpu.CompilerParams(use_tc_tiling_on_sc=False,
                                           needs_layout_passes=False),
      scratch_types=[pltpu.VMEM((PER_W,), jnp.int32),
                     pltpu.VMEM((PER_W,), jnp.int32),
                     pltpu.VMEM((PER_W, 16), jnp.float32),
                     pltpu.VMEM((PER_W,), jnp.float32),
                     pltpu.SemaphoreType.DMA],
  )(flat_idx, row_idx, emb1_rows)


def _tc_body(ecatT, fc, xvnT, E, emb2n, w1n, W1n, W1c, b1, W2, b2, W3, b3,
             Wout, const, out):
  f32 = jnp.float32
  c00 = ((0,), (0,))
  dg = lambda a, b: lax.dot_general(a, b, (c00, ((), ())),
                                    preferred_element_type=f32)
  ec = ecatT[...]
  xv = xvnT[...]
  xnum = dg(E[...], xv)
  h = jnp.maximum(dg(W1n[...], xnum) + dg(W1c[...], ec) + b1[...], 0.0)
  h = jnp.maximum(dg(W2[...], h) + b2[...], 0.0)
  h = jnp.maximum(dg(W3[...], h) + b3[...], 0.0)
  deep = dg(Wout[...], h)
  ri = lax.broadcasted_iota(jnp.int32, (CAT * D, D), 0)
  ci = lax.broadcasted_iota(jnp.int32, (CAT * D, D), 1)
  S = jnp.where(ri % D == ci, 1.0, 0.0).astype(f32)
  e2n = emb2n[...]
  s_vec = dg(e2n, xv) + dg(S, ec)
  q_vec = dg(e2n * e2n, xv * xv) + dg(S, ec * ec)
  fm2 = 0.5 * jnp.sum(s_vec * s_vec - q_vec, axis=0, keepdims=True)
  ones = jnp.full((1, CAT), 1.0, dtype=f32)
  fm1 = dg(w1n[...], xv) + lax.dot_general(ones, fc[...], (((1,), (1,)), ((), ())),
                                           preferred_element_type=f32)
  out[...] = const[...] + fm1 + fm2 + deep


def _tc_dense(ecatT, fc, xvnT, E, emb2n, w1n, W1n, W1c, b1, W2, b2, W3, b3,
              Wout, const, nb):
  grid = (nb // BB,)
  full = lambda shape: pl.BlockSpec(shape, lambda i: (0, 0))
  return pl.pallas_call(
      _tc_body,
      grid=grid,
      in_specs=[
          pl.BlockSpec((CAT * D, BB), lambda i: (0, i)),
          pl.BlockSpec((BB, CAT), lambda i: (i, 0)),
          pl.BlockSpec((NUM, BB), lambda i: (0, i)),
          full((NUM, NUM * D)),
          full((NUM, D)),
          full((NUM, 1)),
          full((NUM * D, H)),
          full((CAT * D, H)),
          full((H, 1)),
          full((H, H)),
          full((H, 1)),
          full((H, H)),
          full((H, 1)),
          full((H, 1)),
          full((1, 1)),
      ],
      out_specs=pl.BlockSpec((1, BB), lambda i: (0, i)),
      out_shape=jax.ShapeDtypeStruct((1, nb), jnp.float32),
  )(ecatT, fc, xvnT, E, emb2n, w1n, W1n, W1c, b1, W2, b2, W3, b3, Wout, const)


def kernel(Xi, Xv, w1_num, emb1_cat, emb2_num, emb2_cat, W1, b1, W2, b2, W3,
           b3, Wout, bout, bias):
  f32 = jnp.float32
  idx = Xi[:, :, 0].astype(jnp.int32)
  fdbase = (jnp.arange(CAT, dtype=jnp.int32)[:, None, None] * D
            + jnp.arange(D, dtype=jnp.int32)[None, None, :]) * VG
  qidx = (fdbase + (idx.T[:, :, None] >> 4)).reshape(CAT, B * D)
  lanes = (idx.T & 15)
  t2 = emb2_cat.transpose(0, 2, 1).reshape(CAT * D * VG, 16)

  flat_idx = (idx + jnp.arange(CAT, dtype=jnp.int32)[None, :] * VOCAB).reshape(-1)
  emb1_rows = emb1_cat.reshape(CAT * VOCAB // 16, 16)
  w = _sc_gather_e1(flat_idx, flat_idx >> 4, emb1_rows)
  fc = w.reshape(B, CAT)

  xvnT = Xv[:, :NUM].T
  E = (jnp.eye(NUM, dtype=f32)[:, :, None] * emb2_num[None, :, :]).reshape(NUM, NUM * D)
  W1n = W1[:NUM * D, :]
  W1c = W1[NUM * D:, :]
  const = (bias[0] + bout[0]).reshape(1, 1)

  H2 = B // 2
  ecatT_h = [_sc_gather_e2(qidx[:, h * H2 * D:(h + 1) * H2 * D],
                           lanes[:, h * H2:(h + 1) * H2], t2, H2)
             for h in (0, 1)]
  outs = [_tc_dense(ecatT_h[h], fc[h * H2:(h + 1) * H2],
                    xvnT[:, h * H2:(h + 1) * H2], E, emb2_num, w1_num, W1n,
                    W1c, b1.reshape(H, 1), W2, b2.reshape(H, 1), W3,
                    b3.reshape(H, 1), Wout, const, H2)
          for h in (0, 1)]
  return jnp.concatenate([outs[0][0], outs[1][0]])


# speedup vs baseline: 1.2585x; 1.2585x over previous
import jax
import jax.numpy as jnp
from jax import lax
from jax.experimental import pallas as pl
from jax.experimental.pallas import tpu as pltpu
from jax.experimental.pallas import tpu_sc as plsc

B = 4096
NUM = 13
CAT = 26
FIELDS = 39
VOCAB = 100000
D = 16
H = 400
VG = VOCAB // 16

NC, NS = 2, 16
NW = NC * NS
BPW = B // NW
BCAT = B * CAT
PER_W = BCAT // NW
RPF = BPW * D

BB = 2048


def _sc_gather_e2(qidx, lanes, t2, nb):
  mesh = plsc.VectorSubcoreMesh(core_axis_name="c", subcore_axis_name="s",
                                num_cores=NC, num_subcores=NS)
  bpw = nb // NW
  rpf = bpw * D

  def body(qidx_hbm, lane_hbm, t2_hbm, ecatT_hbm, qidx_v, lane_v, w16, colbuf,
           sem0, sem1):
    li = lax.iota(jnp.int32, 16)
    wid = lax.axis_index("s") * NC + lax.axis_index("c")
    b0 = wid * bpw
    sems = (sem0, sem1)

    def fire(f, p):
      pltpu.sync_copy(qidx_hbm.at[f, pl.ds(b0 * D, rpf)],
                      qidx_v.at[pl.ds(p * rpf, rpf)])
      pltpu.sync_copy(lane_hbm.at[f, pl.ds(b0, bpw)],
                      lane_v.at[pl.ds(p * bpw, bpw)])

      def chunk(c, _):
        pltpu.async_copy(t2_hbm.at[qidx_v.at[pl.ds(p * rpf + c * 128, 128)]],
                         w16.at[pl.ds(p * rpf + c * 128, 128)], sems[p])
        return 0

      lax.fori_loop(0, rpf // 128, chunk, 0)

    def drain_sel_out(f, p):
      pltpu.make_async_copy(t2_hbm.at[pl.ds(0, rpf)],
                            w16.at[pl.ds(p * rpf, rpf)], sems[p]).wait()

      def select(t, _):
        lane_chunk = lane_v[pl.ds(p * bpw + t * 16, 16)]
        rbase = p * rpf + 256 * t + li * 16
        for d in range(D):
          vals = plsc.load_gather(w16, [rbase + d, lane_chunk])
          colbuf[d, pl.ds(t * 16, 16)] = vals
        return 0

      lax.fori_loop(0, bpw // 16, select, 0)
      pltpu.sync_copy(colbuf, ecatT_hbm.at[pl.ds(16 * f, 16), pl.ds(b0, bpw)])

    def pair(k, _):
      f0 = 2 * k
      fire(f0, 0)

      @pl.when(k > 0)
      def _():
        drain_sel_out(f0 - 1, 1)

      fire(f0 + 1, 1)
      drain_sel_out(f0, 0)
      return 0

    lax.fori_loop(0, CAT // 2, pair, 0)
    drain_sel_out(CAT - 1, 1)

  return pl.kernel(
      body,
      out_type=jax.ShapeDtypeStruct((CAT * D, nb), jnp.float32),
      mesh=mesh,
      compiler_params=pltpu.CompilerParams(use_tc_tiling_on_sc=False,
                                           needs_layout_passes=False),
      scratch_types=[pltpu.VMEM((2 * rpf,), jnp.int32),
                     pltpu.VMEM((2 * bpw,), jnp.int32),
                     pltpu.VMEM((2 * rpf, 16), jnp.float32),
                     pltpu.VMEM((D, bpw), jnp.float32),
                     pltpu.SemaphoreType.DMA,
                     pltpu.SemaphoreType.DMA],
  )(qidx, lanes, t2)


def _sc_gather_e1(flat_idx, row_idx, emb1_rows):
  mesh = plsc.VectorSubcoreMesh(core_axis_name="c", subcore_axis_name="s",
                                num_cores=NC, num_subcores=NS)
  CH = 128
  NCH = PER_W // CH

  def body(idx_hbm, ridx_hbm, t1_hbm, w_hbm, idx_v, ridx_v, w16_v, w_v, sem):
    wid = lax.axis_index("s") * NC + lax.axis_index("c")
    base = wid * PER_W
    pltpu.sync_copy(idx_hbm.at[pl.ds(base, PER_W)], idx_v)
    pltpu.sync_copy(ridx_hbm.at[pl.ds(base, PER_W)], ridx_v)
    copies = []
    for c in range(NCH):
      sl = pl.ds(c * CH, CH)
      copies.append(pltpu.async_copy(t1_hbm.at[ridx_v.at[sl]], w16_v.at[sl], sem))
    for c in copies:
      c.wait()

    def pick(t, _):
      sl = pl.ds(t * 16, 16)
      lane = idx_v[sl] & 15
      row = t * 16 + lax.iota(jnp.int32, 16)
      w_v[sl] = plsc.load_gather(w16_v, [row, lane])
      return 0

    lax.fori_loop(0, PER_W // 16, pick, 0)
    pltpu.sync_copy(w_v, w_hbm.at[pl.ds(base, PER_W)])

  return pl.kernel(
      body,
      out_type=jax.ShapeDtypeStruct((BCAT,), jnp.float32),
      mesh=mesh,
      compiler_params=pltpu.CompilerParams(use_tc_tiling_on_sc=False,
                                           needs_layout_passes=False),
      scratch_types=[pltpu.VMEM((PER_W,), jnp.int32),
                     pltpu.VMEM((PER_W,), jnp.int32),
                     pltpu.VMEM((PER_W, 16), jnp.float32),
                     pltpu.VMEM((PER_W,), jnp.float32),
                     pltpu.SemaphoreType.DMA],
  )(flat_idx, row_idx, emb1_rows)


def _tc_body(ecatT, fc, xvnT, E, emb2n, w1n, W1n, W1c, b1, W2, b2, W3, b3,
             Wout, const, out):
  f32 = jnp.float32
  c00 = ((0,), (0,))
  dg = lambda a, b: lax.dot_general(a, b, (c00, ((), ())),
                                    preferred_element_type=f32)
  ec = ecatT[...]
  xv = xvnT[...]
  xnum = dg(E[...], xv)
  h = jnp.maximum(dg(W1n[...], xnum) + dg(W1c[...], ec) + b1[...], 0.0)
  h = jnp.maximum(dg(W2[...], h) + b2[...], 0.0)
  h = jnp.maximum(dg(W3[...], h) + b3[...], 0.0)
  deep = dg(Wout[...], h)
  ri = lax.broadcasted_iota(jnp.int32, (CAT * D, D), 0)
  ci = lax.broadcasted_iota(jnp.int32, (CAT * D, D), 1)
  S = jnp.where(ri % D == ci, 1.0, 0.0).astype(f32)
  e2n = emb2n[...]
  s_vec = dg(e2n, xv) + dg(S, ec)
  q_vec = dg(e2n * e2n, xv * xv) + dg(S, ec * ec)
  fm2 = 0.5 * jnp.sum(s_vec * s_vec - q_vec, axis=0, keepdims=True)
  ones = jnp.full((1, CAT), 1.0, dtype=f32)
  fm1 = dg(w1n[...], xv) + lax.dot_general(ones, fc[...], (((1,), (1,)), ((), ())),
                                           preferred_element_type=f32)
  out[...] = const[...] + fm1 + fm2 + deep


def _tc_dense(ecatT, fc, xvnT, E, emb2n, w1n, W1n, W1c, b1, W2, b2, W3, b3,
              Wout, const, nb):
  grid = (nb // BB,)
  full = lambda shape: pl.BlockSpec(shape, lambda i: (0, 0))
  return pl.pallas_call(
      _tc_body,
      grid=grid,
      in_specs=[
          pl.BlockSpec((CAT * D, BB), lambda i: (0, i)),
          pl.BlockSpec((BB, CAT), lambda i: (i, 0)),
          pl.BlockSpec((NUM, BB), lambda i: (0, i)),
          full((NUM, NUM * D)),
          full((NUM, D)),
          full((NUM, 1)),
          full((NUM * D, H)),
          full((CAT * D, H)),
          full((H, 1)),
          full((H, H)),
          full((H, 1)),
          full((H, H)),
          full((H, 1)),
          full((H, 1)),
          full((1, 1)),
      ],
      out_specs=pl.BlockSpec((1, BB), lambda i: (0, i)),
      out_shape=jax.ShapeDtypeStruct((1, nb), jnp.float32),
  )(ecatT, fc, xvnT, E, emb2n, w1n, W1n, W1c, b1, W2, b2, W3, b3, Wout, const)


def kernel(Xi, Xv, w1_num, emb1_cat, emb2_num, emb2_cat, W1, b1, W2, b2, W3,
           b3, Wout, bout, bias):
  f32 = jnp.float32
  idx = Xi[:, :, 0].astype(jnp.int32)
  fdbase = (jnp.arange(CAT, dtype=jnp.int32)[:, None, None] * D
            + jnp.arange(D, dtype=jnp.int32)[None, None, :]) * VG
  qidx = (fdbase + (idx.T[:, :, None] >> 4)).reshape(CAT, B * D)
  lanes = (idx.T & 15)
  t2 = emb2_cat.transpose(0, 2, 1).reshape(CAT * D * VG, 16)

  flat_idx = (idx + jnp.arange(CAT, dtype=jnp.int32)[None, :] * VOCAB).reshape(-1)
  emb1_rows = emb1_cat.reshape(CAT * VOCAB // 16, 16)
  w = _sc_gather_e1(flat_idx, flat_idx >> 4, emb1_rows)
  fc = w.reshape(B, CAT)

  xvnT = Xv[:, :NUM].T
  E = (jnp.eye(NUM, dtype=f32)[:, :, None] * emb2_num[None, :, :]).reshape(NUM, NUM * D)
  W1n = W1[:NUM * D, :]
  W1c = W1[NUM * D:, :]
  const = (bias[0] + bout[0]).reshape(1, 1)

  ecatT = _sc_gather_e2(qidx, lanes, t2, B)
  out = _tc_dense(ecatT, fc, xvnT, E, emb2_num, w1_num, W1n, W1c,
                  b1.reshape(H, 1), W2, b2.reshape(H, 1), W3, b3.reshape(H, 1),
                  Wout, const, B)
  return out[0]
